# baseline (speedup 1.0000x reference)
import functools

import jax
import jax.numpy as jnp
from jax import lax
from jax.experimental import pallas as pl
from jax.experimental.pallas import tpu as pltpu
from jax.experimental.pallas import tpu_sc as plsc

K = 32
C = 128
NCAND = K * C
BN = 1024
TH = 4
RB = 256


def _k1_body(x_ref, w_ref, be_ref, bd_ref, pre_ref, bmt_ref):
    B2 = x_ref.shape[0]
    xc = x_ref[...] - bd_ref[...]
    p = jnp.dot(xc, w_ref[...].T, preferred_element_type=jnp.float32) + be_ref[...]
    p3 = p.reshape(B2, BN // C, C)
    pre_ref[...] = p3
    bmt_ref[...] = jnp.max(p3, axis=2).T


def _k2_body(bmt_ref, bids_ref, t_ref):
    G = bmt_ref.shape[0]
    v = bmt_ref[...]
    giota = lax.broadcasted_iota(jnp.int32, (G, RB), 0)
    kiota = lax.broadcasted_iota(jnp.int32, (K, RB), 0)

    def body(i, carry):
        v, bids, _ = carry
        m = jnp.max(v, axis=0, keepdims=True)
        g = jnp.min(jnp.where(v == m, giota, G), axis=0, keepdims=True)
        bids = jnp.where(kiota == i, g, bids)
        v = jnp.where(giota == g, -jnp.inf, v)
        return v, bids, m

    v, bids, m = lax.fori_loop(
        0, K, body,
        (v, jnp.zeros((K, RB), jnp.int32), jnp.zeros((1, RB), jnp.float32)))
    bids_ref[...] = bids.T
    t_ref[...] = m.T


def _sc_body(preG, bids_hbm, tval_hbm, wdec_hbm, bdec_hbm, out_hbm,
             bidv0, fbv0, candv0, actsv0, idxv0, rowsv0,
             bidv1, fbv1, candv1, actsv1, idxv1, rowsv1,
             survv, survi, bdecv, tvv, orow,
             semc0, semc1, semw0, semw1):
    S_TOK = preG.shape[0] // out_hbm.shape[0]
    d = out_hbm.shape[1]
    NW = 32
    TPW = out_hbm.shape[0] // NW
    wid = lax.axis_index("s") * 2 + lax.axis_index("c")
    base = wid * TPW

    pltpu.sync_copy(tval_hbm.at[pl.ds(base, TPW)], tvv)
    pltpu.sync_copy(bdec_hbm, bdecv)
    lane = lax.iota(jnp.int32, 16)
    zero16i = jnp.zeros((16,), jnp.int32)
    neginf = jnp.full((16,), -jnp.inf, jnp.float32)

    def splat(ref, i):
        return plsc.load_gather(ref, [zero16i + i])

    def fetch(ti, bidv, fbv, candv, semc):
        t = base + ti
        pltpu.sync_copy(bids_hbm.at[t], bidv)
        b_lo = bidv[pl.ds(0, 16)]
        b_hi = bidv[pl.ds(16, 16)]
        fbv[pl.ds(0, 16)] = b_lo + t * S_TOK
        fbv[pl.ds(16, 16)] = b_hi + t * S_TOK
        pltpu.async_copy(preG.at[fbv], candv, semc)

    def select(ti, bidv, candv, actsv, idxv, rowsv, semc, semw):
        pltpu.make_async_copy(preG.at[pl.ds(0, K)], candv, semc).wait()
        T = splat(tvv, ti)

        def filt(k, cnt):
            gbase = splat(bidv, k) * C
            for h in range(C // 16):
                vals = candv[k, pl.ds(h * 16, 16)]
                msk = vals >= T

                def slow(cnt=cnt, vals=vals, msk=msk, h=h, gbase=gbase):
                    cs = plsc.cumsum(msk.astype(jnp.int32))
                    pos = cnt + cs - 1
                    plsc.store_scatter(survv, [pos], vals, mask=msk)
                    plsc.store_scatter(survi, [pos], gbase + h * 16 + lane,
                                       mask=msk)
                    return cnt + cs[15]

                cnt = lax.cond(jnp.any(msk), slow, lambda cnt=cnt: cnt)
            return cnt

        cnt = lax.fori_loop(0, K, filt, jnp.int32(0), unroll=False)
        survv[pl.ds(cnt, 16)] = neginf
        survi[pl.ds(cnt, 16)] = zero16i
        nv = (cnt + 15) >> 4

        def ext(i, carry):
            alo, ahi, ilo, ihi = carry

            def mx(v, acc):
                return jnp.maximum(acc, survv[pl.ds(v * 16, 16)])

            acc = lax.fori_loop(0, nv, mx, neginf)
            m = jnp.max(acc)

            def fnd(v, pacc):
                vals = survv[pl.ds(v * 16, 16)]
                return jnp.minimum(pacc, jnp.where(vals == m, lane + v * 16,
                                                   NCAND + 16))

            p = jnp.min(lax.fori_loop(0, nv, fnd, jnp.full((16,), NCAND + 16,
                                                           jnp.int32)))
            plsc.store_scatter(survv, [zero16i + p], neginf, mask=lane == 0)
            g = splat(survi, p)
            msplat = jnp.zeros((16,), jnp.float32) + m
            alo = jnp.where(lane == i, msplat, alo)
            ahi = jnp.where(lane == i - 16, msplat, ahi)
            ilo = jnp.where(lane == i, g, ilo)
            ihi = jnp.where(lane == i - 16, g, ihi)
            return alo, ahi, ilo, ihi

        z16f = jnp.zeros((16,), jnp.float32)
        alo, ahi, ilo, ihi = lax.fori_loop(0, K, ext, (z16f, z16f, zero16i,
                                                       zero16i))
        actsv[pl.ds(0, 16)] = alo
        actsv[pl.ds(16, 16)] = ahi
        idxv[pl.ds(0, 16)] = ilo
        idxv[pl.ds(16, 16)] = ihi
        pltpu.async_copy(wdec_hbm.at[idxv], rowsv, semw)

    def decode(ti, actsv, rowsv, semw):
        pltpu.make_async_copy(wdec_hbm.at[pl.ds(0, K)], rowsv, semw).wait()
        for chunk in range(d // 256):
            accs = [bdecv[pl.ds(chunk * 256 + j * 16, 16)] for j in range(16)]

            def dec(k, accs):
                a = splat(actsv, k)
                return tuple(
                    accs[j] + a * rowsv[k, pl.ds(chunk * 256 + j * 16, 16)]
                    for j in range(16))

            accs = lax.fori_loop(0, K, dec, tuple(accs))
            for j in range(16):
                orow[pl.ds(chunk * 256 + j * 16, 16)] = accs[j]
        pltpu.sync_copy(orow, out_hbm.at[base + ti])

    buf0 = (bidv0, fbv0, candv0, actsv0, idxv0, rowsv0, semc0, semw0)
    buf1 = (bidv1, fbv1, candv1, actsv1, idxv1, rowsv1, semc1, semw1)

    def half(ti, cur, nxt):
        tn = jnp.minimum(ti + 1, TPW - 1)
        fetch(tn, nxt[0], nxt[1], nxt[2], nxt[6])
        select(ti, cur[0], cur[2], cur[3], cur[4], cur[5], cur[6], cur[7])

        @pl.when(ti > 0)
        def _():
            decode(ti - 1, nxt[3], nxt[5], nxt[7])

    fetch(jnp.int32(0), bidv0, fbv0, candv0, semc0)

    def pair(i, _):
        half(i * 2, buf0, buf1)
        half(i * 2 + 1, buf1, buf0)
        return 0

    lax.fori_loop(0, TPW // 2, pair, 0, unroll=False)
    pltpu.make_async_copy(preG.at[pl.ds(0, K)], candv0, semc0).wait()
    decode(TPW - 1, actsv1, rowsv1, semw1)


def kernel(x, W_enc, b_enc, W_dec, b_dec):
    B, d = x.shape
    L = W_enc.shape[0]
    G = L // C

    S = L // 128
    pre3, bmt = pl.pallas_call(
        _k1_body,
        grid=(TH, L // BN),
        in_specs=[
            pl.BlockSpec((B // TH, d), lambda t, j: (t, 0)),
            pl.BlockSpec((BN, d), lambda t, j: (j, 0)),
            pl.BlockSpec((BN,), lambda t, j: (j,)),
            pl.BlockSpec((d,), lambda t, j: (0,)),
        ],
        out_specs=[
            pl.BlockSpec((B // TH, BN // 128, 128), lambda t, j: (t, j, 0)),
            pl.BlockSpec((BN // C, B // TH), lambda t, j: (j, t)),
        ],
        out_shape=[
            jax.ShapeDtypeStruct((B, S, 128), jnp.float32),
            jax.ShapeDtypeStruct((G, B), jnp.float32),
        ],
    )(x, W_enc, b_enc, b_dec)

    bids, tval = pl.pallas_call(
        _k2_body,
        grid=(B // RB,),
        in_specs=[pl.BlockSpec((G, RB), lambda i: (0, i))],
        out_specs=[
            pl.BlockSpec((RB, K), lambda i: (i, 0)),
            pl.BlockSpec((RB, 1), lambda i: (i, 0)),
        ],
        out_shape=[
            jax.ShapeDtypeStruct((B, K), jnp.int32),
            jax.ShapeDtypeStruct((B, 1), jnp.float32),
        ],
    )(bmt)

    sc = functools.partial(
        pl.kernel,
        mesh=plsc.VectorSubcoreMesh(core_axis_name="c", subcore_axis_name="s"),
        out_type=jax.ShapeDtypeStruct((B, d), jnp.float32),
        compiler_params=pltpu.CompilerParams(
            needs_layout_passes=False, use_tc_tiling_on_sc=False),
        scratch_types=(
            [
                pltpu.VMEM((K,), jnp.int32),
                pltpu.VMEM((K,), jnp.int32),
                pltpu.VMEM((K, 128), jnp.float32),
                pltpu.VMEM((K,), jnp.float32),
                pltpu.VMEM((K,), jnp.int32),
                pltpu.VMEM((K, d), jnp.float32),
            ] * 2
            + [
                pltpu.VMEM((NCAND + 16,), jnp.float32),
                pltpu.VMEM((NCAND + 16,), jnp.int32),
                pltpu.VMEM((d,), jnp.float32),
                pltpu.VMEM((B // 32,), jnp.float32),
                pltpu.VMEM((d,), jnp.float32),
            ]
            + [pltpu.SemaphoreType.DMA] * 4
        ),
    )(_sc_body)
    out = sc(pre3.reshape(B * S, 128), bids, tval.reshape(B), W_dec, b_dec)
    return out


# speedup vs baseline: 1.3388x; 1.3388x over previous
import functools

import jax
import jax.numpy as jnp
from jax import lax
from jax.experimental import pallas as pl
from jax.experimental.pallas import tpu as pltpu
from jax.experimental.pallas import tpu_sc as plsc

K = 32
C = 128
NCAND = K * C
BN = 1024
TH = 4
RB = 256


def _k1_body(x_ref, w_ref, be_ref, bd_ref, pre_ref, bmt_ref):
    B2 = x_ref.shape[0]
    xc = x_ref[...] - bd_ref[...]
    p = jnp.dot(xc, w_ref[...].T, preferred_element_type=jnp.float32) + be_ref[...]
    p3 = p.reshape(B2, BN // C, C)
    pre_ref[...] = p3
    bmt_ref[...] = jnp.max(p3, axis=2).T


def _k2_body(bmt_ref, bids_ref, t_ref):
    G = bmt_ref.shape[0]
    v = bmt_ref[...]
    giota = lax.broadcasted_iota(jnp.int32, (G, RB), 0)
    kiota = lax.broadcasted_iota(jnp.int32, (K, RB), 0)

    def body(i, carry):
        v, bids, _ = carry
        m = jnp.max(v, axis=0, keepdims=True)
        g = jnp.min(jnp.where(v == m, giota, G), axis=0, keepdims=True)
        bids = jnp.where(kiota == i, g, bids)
        v = jnp.where(giota == g, -jnp.inf, v)
        return v, bids, m

    v, bids, m = lax.fori_loop(
        0, K, body,
        (v, jnp.zeros((K, RB), jnp.int32), jnp.zeros((1, RB), jnp.float32)))
    bids_ref[...] = bids.T
    t_ref[...] = m.T


def _sc_body(preG, bids_hbm, tval_hbm, wdec_hbm, bdec_hbm, out_hbm,
             bidv0, fbv0, candv0, actsv0, idxv0, rowsv0,
             bidv1, fbv1, candv1, actsv1, idxv1, rowsv1,
             survv, survi, bdecv, tvv, orow,
             semc0, semc1, semw0, semw1):
    S_TOK = preG.shape[0] // out_hbm.shape[0]
    d = out_hbm.shape[1]
    NW = 32
    TPW = out_hbm.shape[0] // NW
    wid = lax.axis_index("s") * 2 + lax.axis_index("c")
    base = wid * TPW

    pltpu.sync_copy(tval_hbm.at[pl.ds(base, TPW)], tvv)
    pltpu.sync_copy(bdec_hbm, bdecv)
    lane = lax.iota(jnp.int32, 16)
    zero16i = jnp.zeros((16,), jnp.int32)
    neginf = jnp.full((16,), -jnp.inf, jnp.float32)

    def splat(ref, i):
        return plsc.load_gather(ref, [zero16i + i])

    def fetch(ti, bidv, fbv, candv, semc):
        t = base + ti
        pltpu.sync_copy(bids_hbm.at[t], bidv)
        b_lo = bidv[pl.ds(0, 16)]
        b_hi = bidv[pl.ds(16, 16)]
        fbv[pl.ds(0, 16)] = b_lo + t * S_TOK
        fbv[pl.ds(16, 16)] = b_hi + t * S_TOK
        pltpu.async_copy(preG.at[fbv], candv, semc)

    def select(ti, bidv, candv, actsv, idxv, rowsv, semc, semw):
        pltpu.make_async_copy(preG.at[pl.ds(0, K)], candv, semc).wait()
        T = splat(tvv, ti)

        def filt(k, cnt):
            gbase = splat(bidv, k) * C
            for h in range(C // 16):
                vals = candv[k, pl.ds(h * 16, 16)]
                msk = vals >= T
                cs = plsc.cumsum(msk.astype(jnp.int32))
                pos = cnt + cs - 1
                plsc.store_scatter(survv, [pos], vals, mask=msk)
                plsc.store_scatter(survi, [pos], gbase + h * 16 + lane,
                                   mask=msk)
                cnt = cnt + cs[15]
            return cnt

        cnt = lax.fori_loop(0, K, filt, jnp.int32(0), unroll=False)
        survv[pl.ds(cnt, 16)] = neginf
        survi[pl.ds(cnt, 16)] = zero16i
        nv = (cnt + 15) >> 4

        def ext(i, carry):
            alo, ahi, ilo, ihi = carry

            def mx(v, acc):
                return jnp.maximum(acc, survv[pl.ds(v * 16, 16)])

            acc = lax.fori_loop(0, nv, mx, neginf)
            m = jnp.max(acc)

            def fnd(v, pacc):
                vals = survv[pl.ds(v * 16, 16)]
                return jnp.minimum(pacc, jnp.where(vals == m, lane + v * 16,
                                                   NCAND + 16))

            p = jnp.min(lax.fori_loop(0, nv, fnd, jnp.full((16,), NCAND + 16,
                                                           jnp.int32)))
            plsc.store_scatter(survv, [zero16i + p], neginf, mask=lane == 0)
            g = splat(survi, p)
            msplat = jnp.zeros((16,), jnp.float32) + m
            alo = jnp.where(lane == i, msplat, alo)
            ahi = jnp.where(lane == i - 16, msplat, ahi)
            ilo = jnp.where(lane == i, g, ilo)
            ihi = jnp.where(lane == i - 16, g, ihi)
            return alo, ahi, ilo, ihi

        z16f = jnp.zeros((16,), jnp.float32)
        alo, ahi, ilo, ihi = lax.fori_loop(0, K, ext, (z16f, z16f, zero16i,
                                                       zero16i))
        actsv[pl.ds(0, 16)] = alo
        actsv[pl.ds(16, 16)] = ahi
        idxv[pl.ds(0, 16)] = ilo
        idxv[pl.ds(16, 16)] = ihi
        pltpu.async_copy(wdec_hbm.at[idxv], rowsv, semw)

    def decode(ti, actsv, rowsv, semw):
        pltpu.make_async_copy(wdec_hbm.at[pl.ds(0, K)], rowsv, semw).wait()
        for chunk in range(d // 256):
            accs = [bdecv[pl.ds(chunk * 256 + j * 16, 16)] for j in range(16)]

            def dec(k, accs):
                a = splat(actsv, k)
                return tuple(
                    accs[j] + a * rowsv[k, pl.ds(chunk * 256 + j * 16, 16)]
                    for j in range(16))

            accs = lax.fori_loop(0, K, dec, tuple(accs))
            for j in range(16):
                orow[pl.ds(chunk * 256 + j * 16, 16)] = accs[j]
        pltpu.sync_copy(orow, out_hbm.at[base + ti])

    buf0 = (bidv0, fbv0, candv0, actsv0, idxv0, rowsv0, semc0, semw0)
    buf1 = (bidv1, fbv1, candv1, actsv1, idxv1, rowsv1, semc1, semw1)

    def half(ti, cur, nxt):
        tn = jnp.minimum(ti + 1, TPW - 1)
        fetch(tn, nxt[0], nxt[1], nxt[2], nxt[6])
        select(ti, cur[0], cur[2], cur[3], cur[4], cur[5], cur[6], cur[7])

        @pl.when(ti > 0)
        def _():
            decode(ti - 1, nxt[3], nxt[5], nxt[7])

    fetch(jnp.int32(0), bidv0, fbv0, candv0, semc0)

    def pair(i, _):
        half(i * 2, buf0, buf1)
        half(i * 2 + 1, buf1, buf0)
        return 0

    lax.fori_loop(0, TPW // 2, pair, 0, unroll=False)
    pltpu.make_async_copy(preG.at[pl.ds(0, K)], candv0, semc0).wait()
    decode(TPW - 1, actsv1, rowsv1, semw1)


def kernel(x, W_enc, b_enc, W_dec, b_dec):
    B, d = x.shape
    L = W_enc.shape[0]
    G = L // C

    S = L // 128
    pre3, bmt = pl.pallas_call(
        _k1_body,
        grid=(TH, L // BN),
        in_specs=[
            pl.BlockSpec((B // TH, d), lambda t, j: (t, 0)),
            pl.BlockSpec((BN, d), lambda t, j: (j, 0)),
            pl.BlockSpec((BN,), lambda t, j: (j,)),
            pl.BlockSpec((d,), lambda t, j: (0,)),
        ],
        out_specs=[
            pl.BlockSpec((B // TH, BN // 128, 128), lambda t, j: (t, j, 0)),
            pl.BlockSpec((BN // C, B // TH), lambda t, j: (j, t)),
        ],
        out_shape=[
            jax.ShapeDtypeStruct((B, S, 128), jnp.float32),
            jax.ShapeDtypeStruct((G, B), jnp.float32),
        ],
    )(x, W_enc, b_enc, b_dec)

    bids, tval = pl.pallas_call(
        _k2_body,
        grid=(B // RB,),
        in_specs=[pl.BlockSpec((G, RB), lambda i: (0, i))],
        out_specs=[
            pl.BlockSpec((RB, K), lambda i: (i, 0)),
            pl.BlockSpec((RB, 1), lambda i: (i, 0)),
        ],
        out_shape=[
            jax.ShapeDtypeStruct((B, K), jnp.int32),
            jax.ShapeDtypeStruct((B, 1), jnp.float32),
        ],
    )(bmt)

    sc = functools.partial(
        pl.kernel,
        mesh=plsc.VectorSubcoreMesh(core_axis_name="c", subcore_axis_name="s"),
        out_type=jax.ShapeDtypeStruct((B, d), jnp.float32),
        compiler_params=pltpu.CompilerParams(
            needs_layout_passes=False, use_tc_tiling_on_sc=False),
        scratch_types=(
            [
                pltpu.VMEM((K,), jnp.int32),
                pltpu.VMEM((K,), jnp.int32),
                pltpu.VMEM((K, 128), jnp.float32),
                pltpu.VMEM((K,), jnp.float32),
                pltpu.VMEM((K,), jnp.int32),
                pltpu.VMEM((K, d), jnp.float32),
            ] * 2
            + [
                pltpu.VMEM((NCAND + 16,), jnp.float32),
                pltpu.VMEM((NCAND + 16,), jnp.int32),
                pltpu.VMEM((d,), jnp.float32),
                pltpu.VMEM((B // 32,), jnp.float32),
                pltpu.VMEM((d,), jnp.float32),
            ]
            + [pltpu.SemaphoreType.DMA] * 4
        ),
    )(_sc_body)
    out = sc(pre3.reshape(B * S, 128), bids, tval.reshape(B), W_dec, b_dec)
    return out


# speedup vs baseline: 1.3390x; 1.0002x over previous
import functools

import jax
import jax.numpy as jnp
from jax import lax
from jax.experimental import pallas as pl
from jax.experimental.pallas import tpu as pltpu
from jax.experimental.pallas import tpu_sc as plsc

K = 32
C = 128
NCAND = K * C
BN = 1024
TH = 4
RB = 256


def _k1_body(x_ref, w_ref, be_ref, bd_ref, pre_ref, bmt_ref):
    B2 = x_ref.shape[0]
    xc = x_ref[...] - bd_ref[...]
    p = jnp.dot(xc, w_ref[...].T, preferred_element_type=jnp.float32) + be_ref[...]
    p3 = p.reshape(B2, BN // C, C)
    pre_ref[...] = p3
    bmt_ref[...] = jnp.max(p3, axis=2).T


def _k2_body(bmt_ref, bids_ref, t_ref):
    G = bmt_ref.shape[0]
    v = bmt_ref[...]
    giota = lax.broadcasted_iota(jnp.int32, (G, RB), 0)
    kiota = lax.broadcasted_iota(jnp.int32, (K, RB), 0)

    def body(i, carry):
        v, bids, _ = carry
        m = jnp.max(v, axis=0, keepdims=True)
        g = jnp.min(jnp.where(v == m, giota, G), axis=0, keepdims=True)
        bids = jnp.where(kiota == i, g, bids)
        v = jnp.where(giota == g, -jnp.inf, v)
        return v, bids, m

    v, bids, m = lax.fori_loop(
        0, K, body,
        (v, jnp.zeros((K, RB), jnp.int32), jnp.zeros((1, RB), jnp.float32)))
    bids_ref[...] = bids.T
    t_ref[...] = m.T


def _sc_body(preG, bids_hbm, tval_hbm, wdec_hbm, bdec_hbm, out_hbm,
             bidv0, fbv0, candv0, actsv0, idxv0, rowsv0,
             bidv1, fbv1, candv1, actsv1, idxv1, rowsv1,
             survv, survi, bdecv, tvv, orow,
             semc0, semc1, semw0, semw1):
    S_TOK = preG.shape[0] // out_hbm.shape[0]
    d = out_hbm.shape[1]
    NW = 32
    TPW = out_hbm.shape[0] // NW
    wid = lax.axis_index("s") * 2 + lax.axis_index("c")
    base = wid * TPW

    pltpu.sync_copy(tval_hbm.at[pl.ds(base, TPW)], tvv)
    pltpu.sync_copy(bdec_hbm, bdecv)
    lane = lax.iota(jnp.int32, 16)
    zero16i = jnp.zeros((16,), jnp.int32)
    neginf = jnp.full((16,), -jnp.inf, jnp.float32)

    def splat(ref, i):
        return plsc.load_gather(ref, [zero16i + i])

    def fetch(ti, bidv, fbv, candv, semc):
        t = base + ti
        pltpu.sync_copy(bids_hbm.at[t], bidv)
        b_lo = bidv[pl.ds(0, 16)]
        b_hi = bidv[pl.ds(16, 16)]
        fbv[pl.ds(0, 16)] = b_lo + t * S_TOK
        fbv[pl.ds(16, 16)] = b_hi + t * S_TOK
        pltpu.async_copy(preG.at[fbv], candv, semc)

    def select(ti, bidv, candv, actsv, idxv, rowsv, semc, semw):
        pltpu.make_async_copy(preG.at[pl.ds(0, K)], candv, semc).wait()
        T = splat(tvv, ti)

        def filt(k, cnt):
            gbase = splat(bidv, k) * C
            for h in range(C // 16):
                vals = candv[k, pl.ds(h * 16, 16)]
                msk = vals >= T
                cs = plsc.cumsum(msk.astype(jnp.int32))
                pos = cnt + cs - 1
                plsc.store_scatter(survv, [pos], vals, mask=msk)
                plsc.store_scatter(survi, [pos], gbase + h * 16 + lane,
                                   mask=msk)
                cnt = cnt + cs[15]
            return cnt

        cnt = lax.fori_loop(0, K, filt, jnp.int32(0), unroll=False)
        survv[pl.ds(cnt, 16)] = neginf
        survi[pl.ds(cnt, 16)] = zero16i
        nv = (cnt + 15) >> 4

        def ext(i, carry):
            alo, ahi, ilo, ihi = carry

            def mx(v, acc):
                return jnp.maximum(acc, survv[pl.ds(v * 16, 16)])

            acc = lax.fori_loop(0, nv, mx, neginf)
            m = jnp.max(acc)

            def fnd(v, pacc):
                vals = survv[pl.ds(v * 16, 16)]
                return jnp.minimum(pacc, jnp.where(vals == m, lane + v * 16,
                                                   NCAND + 16))

            p = jnp.min(lax.fori_loop(0, nv, fnd, jnp.full((16,), NCAND + 16,
                                                           jnp.int32)))
            plsc.store_scatter(survv, [zero16i + p], neginf, mask=lane == 0)
            g = splat(survi, p)
            msplat = jnp.zeros((16,), jnp.float32) + m
            alo = jnp.where(lane == i, msplat, alo)
            ahi = jnp.where(lane == i - 16, msplat, ahi)
            ilo = jnp.where(lane == i, g, ilo)
            ihi = jnp.where(lane == i - 16, g, ihi)
            return alo, ahi, ilo, ihi

        z16f = jnp.zeros((16,), jnp.float32)
        alo, ahi, ilo, ihi = lax.fori_loop(0, K, ext, (z16f, z16f, zero16i,
                                                       zero16i))
        actsv[pl.ds(0, 16)] = alo
        actsv[pl.ds(16, 16)] = ahi
        idxv[pl.ds(0, 16)] = ilo
        idxv[pl.ds(16, 16)] = ihi
        pltpu.async_copy(wdec_hbm.at[idxv], rowsv, semw)

    def decode(ti, actsv, rowsv, semw):
        pltpu.make_async_copy(wdec_hbm.at[pl.ds(0, K)], rowsv, semw).wait()
        for chunk in range(d // 256):
            accs = [bdecv[pl.ds(chunk * 256 + j * 16, 16)] for j in range(16)]

            def dec(k, accs):
                a = splat(actsv, k)
                return tuple(
                    accs[j] + a * rowsv[k, pl.ds(chunk * 256 + j * 16, 16)]
                    for j in range(16))

            accs = lax.fori_loop(0, K, dec, tuple(accs), unroll=4)
            for j in range(16):
                orow[pl.ds(chunk * 256 + j * 16, 16)] = accs[j]
        pltpu.sync_copy(orow, out_hbm.at[base + ti])

    buf0 = (bidv0, fbv0, candv0, actsv0, idxv0, rowsv0, semc0, semw0)
    buf1 = (bidv1, fbv1, candv1, actsv1, idxv1, rowsv1, semc1, semw1)

    def half(ti, cur, nxt):
        tn = jnp.minimum(ti + 1, TPW - 1)
        fetch(tn, nxt[0], nxt[1], nxt[2], nxt[6])
        select(ti, cur[0], cur[2], cur[3], cur[4], cur[5], cur[6], cur[7])

        @pl.when(ti > 0)
        def _():
            decode(ti - 1, nxt[3], nxt[5], nxt[7])

    fetch(jnp.int32(0), bidv0, fbv0, candv0, semc0)

    def pair(i, _):
        half(i * 2, buf0, buf1)
        half(i * 2 + 1, buf1, buf0)
        return 0

    lax.fori_loop(0, TPW // 2, pair, 0, unroll=False)
    pltpu.make_async_copy(preG.at[pl.ds(0, K)], candv0, semc0).wait()
    decode(TPW - 1, actsv1, rowsv1, semw1)


def kernel(x, W_enc, b_enc, W_dec, b_dec):
    B, d = x.shape
    L = W_enc.shape[0]
    G = L // C

    S = L // 128
    pre3, bmt = pl.pallas_call(
        _k1_body,
        grid=(TH, L // BN),
        in_specs=[
            pl.BlockSpec((B // TH, d), lambda t, j: (t, 0)),
            pl.BlockSpec((BN, d), lambda t, j: (j, 0)),
            pl.BlockSpec((BN,), lambda t, j: (j,)),
            pl.BlockSpec((d,), lambda t, j: (0,)),
        ],
        out_specs=[
            pl.BlockSpec((B // TH, BN // 128, 128), lambda t, j: (t, j, 0)),
            pl.BlockSpec((BN // C, B // TH), lambda t, j: (j, t)),
        ],
        out_shape=[
            jax.ShapeDtypeStruct((B, S, 128), jnp.float32),
            jax.ShapeDtypeStruct((G, B), jnp.float32),
        ],
    )(x, W_enc, b_enc, b_dec)

    bids, tval = pl.pallas_call(
        _k2_body,
        grid=(B // RB,),
        in_specs=[pl.BlockSpec((G, RB), lambda i: (0, i))],
        out_specs=[
            pl.BlockSpec((RB, K), lambda i: (i, 0)),
            pl.BlockSpec((RB, 1), lambda i: (i, 0)),
        ],
        out_shape=[
            jax.ShapeDtypeStruct((B, K), jnp.int32),
            jax.ShapeDtypeStruct((B, 1), jnp.float32),
        ],
    )(bmt)

    sc = functools.partial(
        pl.kernel,
        mesh=plsc.VectorSubcoreMesh(core_axis_name="c", subcore_axis_name="s"),
        out_type=jax.ShapeDtypeStruct((B, d), jnp.float32),
        compiler_params=pltpu.CompilerParams(
            needs_layout_passes=False, use_tc_tiling_on_sc=False),
        scratch_types=(
            [
                pltpu.VMEM((K,), jnp.int32),
                pltpu.VMEM((K,), jnp.int32),
                pltpu.VMEM((K, 128), jnp.float32),
                pltpu.VMEM((K,), jnp.float32),
                pltpu.VMEM((K,), jnp.int32),
                pltpu.VMEM((K, d), jnp.float32),
            ] * 2
            + [
                pltpu.VMEM((NCAND + 16,), jnp.float32),
                pltpu.VMEM((NCAND + 16,), jnp.int32),
                pltpu.VMEM((d,), jnp.float32),
                pltpu.VMEM((B // 32,), jnp.float32),
                pltpu.VMEM((d,), jnp.float32),
            ]
            + [pltpu.SemaphoreType.DMA] * 4
        ),
    )(_sc_body)
    out = sc(pre3.reshape(B * S, 128), bids, tval.reshape(B), W_dec, b_dec)
    return out
